# baseline (speedup 1.0000x reference)
import functools

import jax
import jax.numpy as jnp
from jax import lax
from jax.experimental import pallas as pl
from jax.experimental.pallas import tpu as pltpu
from jax.experimental.pallas import tpu_sc as plsc

N = 10000
E = 320000
D_IN = 128
D_H = 64
D_E = 32
K = 10

NC = 2
NS = 16
NW = NC * NS
L = 16
BL = 128
NBLOCKS = E // BL
NBLK = NBLOCKS // NW
NEXTRA = NBLOCKS - NW * NBLK
EPW = NBLK * BL
ZR = 624
ZR_LAST = N - 15 * ZR

_mesh = plsc.VectorSubcoreMesh(core_axis_name="c", subcore_axis_name="s")


def _tc_a_body(x_ref, wg_ref, al_ref, ar_ref, feat_ref, el_ref, er_ref,
               m_ref):
    x = x_ref[...]
    feat = lax.dot_general(x, wg_ref[...], (((1,), (1,)), ((), ())),
                           preferred_element_type=jnp.float32)
    feat_ref[...] = feat
    el = lax.dot_general(al_ref[...], feat, (((1,), (1,)), ((), ())),
                         preferred_element_type=jnp.float32)
    er = lax.dot_general(ar_ref[...], feat, (((1,), (1,)), ((), ())),
                         preferred_element_type=jnp.float32)
    el_ref[...] = el[0]
    er_ref[...] = er[0]
    m = jnp.max(el) + jnp.max(er)
    m = jnp.where(m < 0.0, 0.2 * m, m)
    m_ref[...] = jnp.broadcast_to(m, (L,))


_tc_a = pl.pallas_call(
    _tc_a_body,
    out_shape=[
        jax.ShapeDtypeStruct((N, D_H), jnp.float32),
        jax.ShapeDtypeStruct((N,), jnp.float32),
        jax.ShapeDtypeStruct((N,), jnp.float32),
        jax.ShapeDtypeStruct((L,), jnp.float32),
    ],
)


@functools.partial(
    pl.kernel,
    out_type=[
        jax.ShapeDtypeStruct((E,), jnp.float32),
        jax.ShapeDtypeStruct((NW * N,), jnp.float32),
    ],
    mesh=_mesh,
    compiler_params=pltpu.CompilerParams(needs_layout_passes=False,
                                         use_tc_tiling_on_sc=False),
    scratch_types=[
        pltpu.VMEM((N,), jnp.float32),
        pltpu.VMEM((N,), jnp.float32),
        pltpu.VMEM((NBLK, 2, BL), jnp.int32),
        pltpu.VMEM((2, BL), jnp.int32),
        pltpu.VMEM((EPW,), jnp.float32),
        pltpu.VMEM((BL,), jnp.float32),
        pltpu.VMEM((N,), jnp.float32),
        pltpu.VMEM((L,), jnp.float32),
        pltpu.SemaphoreType.DMA,
    ],
)
def _sc_attn(ei_hbm, el_hbm, er_hbm, m_hbm, ee_out, den_out,
             el_v, er_v, ei_v, exv, ee_v, exee, den_v, m_v, stsem):
    c = lax.axis_index("c")
    s = lax.axis_index("s")
    wid = s * NC + c
    bstart = wid * NBLK
    d1 = pltpu.async_copy(el_hbm, el_v, stsem)
    d2 = pltpu.async_copy(er_hbm, er_v, stsem)
    d3 = pltpu.async_copy(ei_hbm.at[pl.ds(bstart, NBLK)], ei_v, stsem)
    d5 = pltpu.async_copy(m_hbm, m_v, stsem)
    zero = jnp.zeros((L,), jnp.float32)

    def zbody(i, carry):
        den_v[pl.ds(i * L, L)] = zero
        return carry

    lax.fori_loop(0, N // L, zbody, 0)
    d1.wait()
    d2.wait()
    d3.wait()
    d5.wait()
    m16 = m_v[...]

    def edge16(src_ref, dst_ref, ee_ref, t, k, eoff):
        s16 = src_ref[t, 0, pl.ds(k * L, L)]
        d16 = dst_ref[t, 1, pl.ds(k * L, L)]
        e = plsc.load_gather(el_v, [s16]) + plsc.load_gather(er_v, [d16])
        e = jnp.where(e < 0.0, e * jnp.float32(0.2), e)
        ee = jnp.exp(e - m16)
        ee_ref[pl.ds(eoff, L)] = ee
        plsc.addupdate_scatter(den_v, [d16], ee)

    def tbody(t, carry):
        def kbody(k, carry2):
            edge16(ei_v, ei_v, ee_v, t, k, t * BL + k * L)
            return carry2

        return lax.fori_loop(0, BL // L, kbody, carry)

    lax.fori_loop(0, NBLK, tbody, 0)
    pltpu.sync_copy(ee_v, ee_out.at[pl.ds(bstart * BL, EPW)])

    @pl.when(wid < NEXTRA)
    def _():
        xb = NW * NBLK + wid
        pltpu.async_copy(ei_hbm.at[xb], exv, stsem).wait()

        def xbody(k, carry):
            s16 = exv[0, pl.ds(k * L, L)]
            d16 = exv[1, pl.ds(k * L, L)]
            e = (plsc.load_gather(el_v, [s16])
                 + plsc.load_gather(er_v, [d16]))
            e = jnp.where(e < 0.0, e * jnp.float32(0.2), e)
            ee = jnp.exp(e - m16)
            exee[pl.ds(k * L, L)] = ee
            plsc.addupdate_scatter(den_v, [d16], ee)
            return carry

        lax.fori_loop(0, BL // L, xbody, 0)
        pltpu.sync_copy(exee, ee_out.at[pl.ds(xb * BL, BL)])

    pltpu.sync_copy(den_v, den_out.at[pl.ds(wid * N, N)])


def _tc_red_body(parts_ref, den_ref):
    acc = parts_ref[pl.ds(0, N)]
    for p in range(1, NW):
        acc = acc + parts_ref[pl.ds(p * N, N)]
    den_ref[...] = acc


_tc_red = pl.pallas_call(
    _tc_red_body,
    out_shape=jax.ShapeDtypeStruct((N,), jnp.float32),
)


NB = 3
NGRP = NBLK // NB


@functools.partial(
    pl.kernel,
    out_type=jax.ShapeDtypeStruct((NC, N, D_H), jnp.float32),
    mesh=_mesh,
    compiler_params=pltpu.CompilerParams(needs_layout_passes=False,
                                         use_tc_tiling_on_sc=False),
    scratch_types=[
        pltpu.VMEM((NBLK, 2, BL), jnp.int32),
        pltpu.VMEM((2, BL), jnp.int32),
        pltpu.VMEM((N,), jnp.float32),
        pltpu.VMEM((BL,), jnp.float32),
        tuple(pltpu.VMEM((BL, D_H), jnp.float32) for _ in range(NB)),
        tuple(pltpu.VMEM((BL, D_H), jnp.float32) for _ in range(NB)),
        tuple(pltpu.VMEM((BL,), jnp.float32) for _ in range(NB)),
        pltpu.VMEM((BL,), jnp.float32),
        pltpu.VMEM_SHARED((N, D_H), jnp.float32),
        tuple(pltpu.SemaphoreType.DMA for _ in range(NB)),
        tuple(pltpu.SemaphoreType.DMA for _ in range(NB)),
        pltpu.SemaphoreType.DMA,
    ],
)
def _sc_aggr(ei_hbm, ee_hbm, den_hbm, feat_hbm, out_hbm,
             ei_v, exv, den_v, exee, gbuf, sbuf, ebuf, alpha_v,
             rst_sh, gsem, ssem, stsem):
    zbuf = sbuf[0]
    c = lax.axis_index("c")
    s = lax.axis_index("s")
    wid = s * NC + c
    bstart = wid * NBLK
    d_ei = pltpu.async_copy(ei_hbm.at[pl.ds(bstart, NBLK)], ei_v, stsem)
    d_den = pltpu.async_copy(den_hbm, den_v, stsem)

    zero = jnp.zeros((L,), jnp.float32)

    def zrow(r, carry):
        def zcol(cc, carry2):
            zbuf[r, pl.ds(cc * L, L)] = zero
            return carry2

        return lax.fori_loop(0, D_H // L, zcol, carry)

    lax.fori_loop(0, BL, zrow, 0)

    @pl.when(s < NS - 1)
    def _():
        def zc(k, carry):
            pltpu.sync_copy(zbuf, rst_sh.at[pl.ds(s * ZR + k * BL, BL)])
            return carry

        lax.fori_loop(0, ZR // BL, zc, 0)
        pltpu.sync_copy(zbuf.at[pl.ds(0, ZR % BL)],
                        rst_sh.at[pl.ds(s * ZR + (ZR // BL) * BL,
                                        ZR % BL)])

    @pl.when(s == NS - 1)
    def _():
        def zc(k, carry):
            pltpu.sync_copy(zbuf,
                            rst_sh.at[pl.ds((NS - 1) * ZR + k * BL, BL)])
            return carry

        lax.fori_loop(0, ZR_LAST // BL, zc, 0)

    d_ei.wait()
    d_den.wait()
    plsc.subcore_barrier()

    def issue_gathers(j, b):
        pltpu.async_copy(feat_hbm.at[ei_v.at[j, 0]], gbuf[b], gsem[b])
        pltpu.async_copy(ee_hbm.at[pl.ds((bstart + j) * BL, BL)], ebuf[b],
                         gsem[b])

    for b in range(NB):
        issue_gathers(b, b)

    def process(j, b, wait_prev_scatter):
        pltpu.make_async_copy(feat_hbm.at[ei_v.at[j, 0]], gbuf[b],
                              gsem[b]).wait()
        pltpu.make_async_copy(ee_hbm.at[pl.ds((bstart + j) * BL, BL)],
                              ebuf[b], gsem[b]).wait()

        def av(k, carry2):
            d16 = ei_v[j, 1, pl.ds(k * L, L)]
            dn = plsc.load_gather(den_v, [d16])
            ee16 = ebuf[b][pl.ds(k * L, L)]
            alpha_v[pl.ds(k * L, L)] = ee16 / (dn + jnp.float32(1e-16))
            return carry2

        lax.fori_loop(0, BL // L, av, 0)

        if wait_prev_scatter:
            pltpu.make_async_copy(
                sbuf[b], rst_sh.at[ei_v.at[j, 1]], ssem[b]).wait()

        def se(k, carry2):
            a16 = alpha_v[pl.ds(k * L, L)]
            for jj in range(L):
                a = a16[jj]
                row = k * L + jj
                for cc in range(D_H // L):
                    sl = pl.ds(cc * L, L)
                    sbuf[b][row, sl] = gbuf[b][row, sl] * a
            return carry2

        lax.fori_loop(0, BL // L, se, 0)
        pltpu.async_copy(sbuf[b], rst_sh.at[ei_v.at[j, 1]], ssem[b],
                         add=True)

        @pl.when(j + NB < NBLK)
        def _():
            issue_gathers(j + NB, b)

    def group0(g, carry):
        for b in range(NB):
            process(g * NB + b, b, wait_prev_scatter=False)
        return carry

    def group(g, carry):
        for b in range(NB):
            process(g * NB + b, b, wait_prev_scatter=True)
        return carry

    lax.fori_loop(0, 1, group0, 0)
    lax.fori_loop(1, NGRP, group, 0)
    for b in range(NB):
        pltpu.make_async_copy(
            sbuf[b], rst_sh.at[ei_v.at[NBLK - NB + b, 1]], ssem[b]).wait()

    @pl.when(wid < NEXTRA)
    def _():
        xb = NW * NBLK + wid
        pltpu.async_copy(ei_hbm.at[xb], exv, stsem).wait()
        pltpu.async_copy(ee_hbm.at[pl.ds(xb * BL, BL)], exee, stsem).wait()
        pltpu.async_copy(feat_hbm.at[exv.at[0]], gbuf[0], stsem).wait()

        def xav(k, carry2):
            d16 = exv[1, pl.ds(k * L, L)]
            dn = plsc.load_gather(den_v, [d16])
            ee16 = exee[pl.ds(k * L, L)]
            alpha_v[pl.ds(k * L, L)] = ee16 / (dn + jnp.float32(1e-16))
            return carry2

        lax.fori_loop(0, BL // L, xav, 0)

        def xse(k, carry2):
            a16 = alpha_v[pl.ds(k * L, L)]
            for jj in range(L):
                a = a16[jj]
                row = k * L + jj
                for cc in range(D_H // L):
                    sl = pl.ds(cc * L, L)
                    sbuf[0][row, sl] = gbuf[0][row, sl] * a
            return carry2

        lax.fori_loop(0, BL // L, xse, 0)
        pltpu.sync_copy(sbuf[0], rst_sh.at[exv.at[1]], add=True)

    plsc.subcore_barrier()

    @pl.when(s < NS - 1)
    def _():
        pltpu.sync_copy(rst_sh.at[pl.ds(s * ZR, ZR)],
                        out_hbm.at[c, pl.ds(s * ZR, ZR)])

    @pl.when(s == NS - 1)
    def _():
        pltpu.sync_copy(rst_sh.at[pl.ds((NS - 1) * ZR, ZR_LAST)],
                        out_hbm.at[c, pl.ds((NS - 1) * ZR, ZR_LAST)])


def _tc_c_body(p_ref, bg_ref, we_ref, wd_ref, cen_ref, embt_ref, pred_ref,
               qt_ref):
    p = p_ref[...]
    rst = p[0] + p[1] + bg_ref[...]
    rst = jnp.where(rst > 0.0, rst, jnp.exp(jnp.minimum(rst, 0.0)) - 1.0)
    embt = lax.dot_general(we_ref[...], rst, (((1,), (1,)), ((), ())),
                           preferred_element_type=jnp.float32)
    embt_ref[...] = embt
    pred = lax.dot_general(embt, wd_ref[...], (((0,), (1,)), ((), ())),
                           preferred_element_type=jnp.float32)
    pred_ref[...] = jnp.where(pred > 0.0, pred,
                              jnp.exp(jnp.minimum(pred, 0.0)) - 1.0)
    cen = cen_ref[...]
    crosst = lax.dot_general(cen, embt, (((1,), (0,)), ((), ())),
                             preferred_element_type=jnp.float32)
    d2t = (jnp.sum(embt * embt, axis=0, keepdims=True) - 2.0 * crosst
           + jnp.sum(cen * cen, axis=1)[:, None])
    qt = 1.0 / (1.0 + d2t + 1e-8)
    qt_ref[...] = qt / jnp.sum(qt, axis=0, keepdims=True)


_tc_c = pl.pallas_call(
    _tc_c_body,
    out_shape=[
        jax.ShapeDtypeStruct((D_E, N), jnp.float32),
        jax.ShapeDtypeStruct((N, D_IN), jnp.float32),
        jax.ShapeDtypeStruct((K, N), jnp.float32),
    ],
)


def kernel(x, edge_index, W_gat, attn_l, attn_r, b_gat, W_enc, W_dec,
           centroids):
    ei3 = edge_index.reshape(2, NBLOCKS, BL).transpose(1, 0, 2)
    feat, el, er, m16 = _tc_a(x, W_gat, attn_l.reshape(1, D_H),
                              attn_r.reshape(1, D_H))
    ee, den_parts = _sc_attn(ei3, el, er, m16)
    denom = _tc_red(den_parts)
    parts = _sc_aggr(ei3, ee, denom, feat)
    embt, pred_gene, qt = _tc_c(parts, b_gat.reshape(1, D_H), W_enc, W_dec,
                                centroids)
    return (embt.T, pred_gene, qt.T)


# speedup vs baseline: 1.0495x; 1.0495x over previous
import functools

import jax
import jax.numpy as jnp
from jax import lax
from jax.experimental import pallas as pl
from jax.experimental.pallas import tpu as pltpu
from jax.experimental.pallas import tpu_sc as plsc

N = 10000
E = 320000
D_IN = 128
D_H = 64
D_E = 32
K = 10

NC = 2
NS = 16
NW = NC * NS
L = 16
BL = 128
NBLOCKS = E // BL
NBLK = NBLOCKS // NW
NEXTRA = NBLOCKS - NW * NBLK
EPW = NBLK * BL
ZR = 624
ZR_LAST = N - 15 * ZR

_mesh = plsc.VectorSubcoreMesh(core_axis_name="c", subcore_axis_name="s")


def _tc_a_body(x_ref, wg_ref, al_ref, ar_ref, feat_ref, el_ref, er_ref,
               m_ref):
    x = x_ref[...]
    feat = lax.dot_general(x, wg_ref[...], (((1,), (1,)), ((), ())),
                           preferred_element_type=jnp.float32)
    feat_ref[...] = feat
    el = lax.dot_general(al_ref[...], feat, (((1,), (1,)), ((), ())),
                         preferred_element_type=jnp.float32)
    er = lax.dot_general(ar_ref[...], feat, (((1,), (1,)), ((), ())),
                         preferred_element_type=jnp.float32)
    el_ref[...] = el[0]
    er_ref[...] = er[0]
    m = jnp.max(el) + jnp.max(er)
    m = jnp.where(m < 0.0, 0.2 * m, m)
    m_ref[...] = jnp.broadcast_to(m, (L,))


_tc_a = pl.pallas_call(
    _tc_a_body,
    out_shape=[
        jax.ShapeDtypeStruct((N, D_H), jnp.float32),
        jax.ShapeDtypeStruct((N,), jnp.float32),
        jax.ShapeDtypeStruct((N,), jnp.float32),
        jax.ShapeDtypeStruct((L,), jnp.float32),
    ],
)


@functools.partial(
    pl.kernel,
    out_type=[
        jax.ShapeDtypeStruct((E,), jnp.float32),
        jax.ShapeDtypeStruct((NW * N,), jnp.float32),
    ],
    mesh=_mesh,
    compiler_params=pltpu.CompilerParams(needs_layout_passes=False,
                                         use_tc_tiling_on_sc=False),
    scratch_types=[
        pltpu.VMEM((N,), jnp.float32),
        pltpu.VMEM((N,), jnp.float32),
        pltpu.VMEM((NBLK, 2, BL), jnp.int32),
        pltpu.VMEM((2, BL), jnp.int32),
        pltpu.VMEM((EPW,), jnp.float32),
        pltpu.VMEM((BL,), jnp.float32),
        pltpu.VMEM((N,), jnp.float32),
        pltpu.VMEM((L,), jnp.float32),
        pltpu.SemaphoreType.DMA,
    ],
)
def _sc_attn(ei_hbm, el_hbm, er_hbm, m_hbm, ee_out, den_out,
             el_v, er_v, ei_v, exv, ee_v, exee, den_v, m_v, stsem):
    c = lax.axis_index("c")
    s = lax.axis_index("s")
    wid = s * NC + c
    bstart = wid * NBLK
    d1 = pltpu.async_copy(el_hbm, el_v, stsem)
    d2 = pltpu.async_copy(er_hbm, er_v, stsem)
    d3 = pltpu.async_copy(ei_hbm.at[pl.ds(bstart, NBLK)], ei_v, stsem)
    d5 = pltpu.async_copy(m_hbm, m_v, stsem)
    zero = jnp.zeros((L,), jnp.float32)

    def zbody(i, carry):
        den_v[pl.ds(i * L, L)] = zero
        return carry

    lax.fori_loop(0, N // L, zbody, 0)
    d1.wait()
    d2.wait()
    d3.wait()
    d5.wait()
    m16 = m_v[...]

    def edge16(src_ref, dst_ref, ee_ref, t, k, eoff):
        s16 = src_ref[t, 0, pl.ds(k * L, L)]
        d16 = dst_ref[t, 1, pl.ds(k * L, L)]
        e = plsc.load_gather(el_v, [s16]) + plsc.load_gather(er_v, [d16])
        e = jnp.where(e < 0.0, e * jnp.float32(0.2), e)
        ee = jnp.exp(e - m16)
        ee_ref[pl.ds(eoff, L)] = ee
        plsc.addupdate_scatter(den_v, [d16], ee)

    def tbody(t, carry):
        def kbody(k, carry2):
            edge16(ei_v, ei_v, ee_v, t, k, t * BL + k * L)
            return carry2

        return lax.fori_loop(0, BL // L, kbody, carry)

    lax.fori_loop(0, NBLK, tbody, 0)
    pltpu.sync_copy(ee_v, ee_out.at[pl.ds(bstart * BL, EPW)])

    @pl.when(wid < NEXTRA)
    def _():
        xb = NW * NBLK + wid
        pltpu.async_copy(ei_hbm.at[xb], exv, stsem).wait()

        def xbody(k, carry):
            s16 = exv[0, pl.ds(k * L, L)]
            d16 = exv[1, pl.ds(k * L, L)]
            e = (plsc.load_gather(el_v, [s16])
                 + plsc.load_gather(er_v, [d16]))
            e = jnp.where(e < 0.0, e * jnp.float32(0.2), e)
            ee = jnp.exp(e - m16)
            exee[pl.ds(k * L, L)] = ee
            plsc.addupdate_scatter(den_v, [d16], ee)
            return carry

        lax.fori_loop(0, BL // L, xbody, 0)
        pltpu.sync_copy(exee, ee_out.at[pl.ds(xb * BL, BL)])

    pltpu.sync_copy(den_v, den_out.at[pl.ds(wid * N, N)])


def _tc_red_body(parts_ref, den_ref):
    acc = parts_ref[pl.ds(0, N)]
    for p in range(1, NW):
        acc = acc + parts_ref[pl.ds(p * N, N)]
    den_ref[...] = acc


_tc_red = pl.pallas_call(
    _tc_red_body,
    out_shape=jax.ShapeDtypeStruct((N,), jnp.float32),
)


NB = 3
NGRP = NBLK // NB


@functools.partial(
    pl.kernel,
    out_type=jax.ShapeDtypeStruct((NC, N, D_H), jnp.float32),
    mesh=_mesh,
    compiler_params=pltpu.CompilerParams(needs_layout_passes=False,
                                         use_tc_tiling_on_sc=False),
    scratch_types=[
        pltpu.VMEM((NBLK, 2, BL), jnp.int32),
        pltpu.VMEM((2, BL), jnp.int32),
        pltpu.VMEM((N,), jnp.float32),
        pltpu.VMEM((BL,), jnp.float32),
        tuple(pltpu.VMEM((BL, D_H), jnp.float32) for _ in range(NB)),
        tuple(pltpu.VMEM((BL, D_H), jnp.float32) for _ in range(NB)),
        tuple(pltpu.VMEM((BL,), jnp.float32) for _ in range(NB)),
        pltpu.VMEM((BL,), jnp.float32),
        pltpu.VMEM_SHARED((N, D_H), jnp.float32),
        tuple(pltpu.SemaphoreType.DMA for _ in range(NB)),
        tuple(pltpu.SemaphoreType.DMA for _ in range(NB)),
        pltpu.SemaphoreType.DMA,
    ],
)
def _sc_aggr(ei_hbm, ee_hbm, den_hbm, feat_hbm, out_hbm,
             ei_v, exv, den_v, exee, gbuf, sbuf, ebuf, alpha_v,
             rst_sh, gsem, ssem, stsem):
    zbuf = sbuf[0]
    c = lax.axis_index("c")
    s = lax.axis_index("s")
    wid = s * NC + c
    bstart = wid * NBLK
    d_ei = pltpu.async_copy(ei_hbm.at[pl.ds(bstart, NBLK)], ei_v, stsem)
    d_den = pltpu.async_copy(den_hbm, den_v, stsem)

    zero = jnp.zeros((L,), jnp.float32)

    def zrow(r, carry):
        def zcol(cc, carry2):
            zbuf[r, pl.ds(cc * L, L)] = zero
            return carry2

        return lax.fori_loop(0, D_H // L, zcol, carry)

    lax.fori_loop(0, BL, zrow, 0)

    @pl.when(s < NS - 1)
    def _():
        def zc(k, carry):
            pltpu.sync_copy(zbuf, rst_sh.at[pl.ds(s * ZR + k * BL, BL)])
            return carry

        lax.fori_loop(0, ZR // BL, zc, 0)
        pltpu.sync_copy(zbuf.at[pl.ds(0, ZR % BL)],
                        rst_sh.at[pl.ds(s * ZR + (ZR // BL) * BL,
                                        ZR % BL)])

    @pl.when(s == NS - 1)
    def _():
        def zc(k, carry):
            pltpu.sync_copy(zbuf,
                            rst_sh.at[pl.ds((NS - 1) * ZR + k * BL, BL)])
            return carry

        lax.fori_loop(0, ZR_LAST // BL, zc, 0)

    d_ei.wait()
    d_den.wait()
    plsc.subcore_barrier()

    def issue_gathers(j, b):
        pltpu.async_copy(feat_hbm.at[ei_v.at[j, 0]], gbuf[b], gsem[b])
        pltpu.async_copy(ee_hbm.at[pl.ds((bstart + j) * BL, BL)], ebuf[b],
                         gsem[b])

    for b in range(NB):
        issue_gathers(b, b)

    def process(j, b, wait_prev_scatter):
        pltpu.make_async_copy(feat_hbm.at[ei_v.at[j, 0]], gbuf[b],
                              gsem[b]).wait()
        pltpu.make_async_copy(ee_hbm.at[pl.ds((bstart + j) * BL, BL)],
                              ebuf[b], gsem[b]).wait()

        def av(k, carry2):
            d16 = ei_v[j, 1, pl.ds(k * L, L)]
            dn = plsc.load_gather(den_v, [d16])
            ee16 = ebuf[b][pl.ds(k * L, L)]
            alpha_v[pl.ds(k * L, L)] = ee16 / (dn + jnp.float32(1e-16))
            return carry2

        lax.fori_loop(0, BL // L, av, 0)

        if wait_prev_scatter:
            pltpu.make_async_copy(
                sbuf[b], rst_sh.at[ei_v.at[j, 1]], ssem[b]).wait()

        def se(k, carry2):
            a16 = alpha_v[pl.ds(k * L, L)]
            for jj in range(L):
                a = a16[jj]
                row = k * L + jj
                for cc in range(D_H // L):
                    sl = pl.ds(cc * L, L)
                    sbuf[b][row, sl] = gbuf[b][row, sl] * a
            return carry2

        lax.fori_loop(0, BL // L, se, 0)
        pltpu.async_copy(sbuf[b], rst_sh.at[ei_v.at[j, 1]], ssem[b],
                         add=True)

        @pl.when(j + NB < NBLK)
        def _():
            issue_gathers(j + NB, b)

    def group0(g, carry):
        for b in range(NB):
            process(g * NB + b, b, wait_prev_scatter=False)
        return carry

    def group(g, carry):
        for b in range(NB):
            process(g * NB + b, b, wait_prev_scatter=True)
        return carry

    lax.fori_loop(0, 1, group0, 0)
    lax.fori_loop(1, NGRP, group, 0)
    for b in range(NB):
        pltpu.make_async_copy(
            sbuf[b], rst_sh.at[ei_v.at[NBLK - NB + b, 1]], ssem[b]).wait()

    @pl.when(wid < NEXTRA)
    def _():
        xb = NW * NBLK + wid
        pltpu.async_copy(ei_hbm.at[xb], exv, stsem).wait()
        pltpu.async_copy(ee_hbm.at[pl.ds(xb * BL, BL)], exee, stsem).wait()
        pltpu.async_copy(feat_hbm.at[exv.at[0]], gbuf[0], stsem).wait()

        def xav(k, carry2):
            d16 = exv[1, pl.ds(k * L, L)]
            dn = plsc.load_gather(den_v, [d16])
            ee16 = exee[pl.ds(k * L, L)]
            alpha_v[pl.ds(k * L, L)] = ee16 / (dn + jnp.float32(1e-16))
            return carry2

        lax.fori_loop(0, BL // L, xav, 0)

        def xse(k, carry2):
            a16 = alpha_v[pl.ds(k * L, L)]
            for jj in range(L):
                a = a16[jj]
                row = k * L + jj
                for cc in range(D_H // L):
                    sl = pl.ds(cc * L, L)
                    sbuf[0][row, sl] = gbuf[0][row, sl] * a
            return carry2

        lax.fori_loop(0, BL // L, xse, 0)
        pltpu.sync_copy(sbuf[0], rst_sh.at[exv.at[1]], add=True)

    plsc.subcore_barrier()

    @pl.when(s < NS - 1)
    def _():
        pltpu.sync_copy(rst_sh.at[pl.ds(s * ZR, ZR)],
                        out_hbm.at[c, pl.ds(s * ZR, ZR)])

    @pl.when(s == NS - 1)
    def _():
        pltpu.sync_copy(rst_sh.at[pl.ds((NS - 1) * ZR, ZR_LAST)],
                        out_hbm.at[c, pl.ds((NS - 1) * ZR, ZR_LAST)])


def _tc_c_body(p_ref, bg_ref, we_ref, wd_ref, cen_ref, embt_ref, pred_ref,
               qt_ref):
    p = p_ref[...]
    rstp = p[0] + p[1] + bg_ref[...]
    rstp = jnp.where(rstp > 0.0, rstp,
                     jnp.exp(jnp.minimum(rstp, 0.0)) - 1.0)
    rst = jnp.stack([rstp[:, 0:D_H], rstp[:, D_H:2 * D_H]],
                    axis=1).reshape(N, D_H)
    embt = lax.dot_general(we_ref[...], rst, (((1,), (1,)), ((), ())),
                           preferred_element_type=jnp.float32)
    embt_ref[...] = embt
    pred = lax.dot_general(embt, wd_ref[...], (((0,), (1,)), ((), ())),
                           preferred_element_type=jnp.float32)
    pred_ref[...] = jnp.where(pred > 0.0, pred,
                              jnp.exp(jnp.minimum(pred, 0.0)) - 1.0)
    cen = cen_ref[...]
    crosst = lax.dot_general(cen, embt, (((1,), (0,)), ((), ())),
                             preferred_element_type=jnp.float32)
    d2t = (jnp.sum(embt * embt, axis=0, keepdims=True) - 2.0 * crosst
           + jnp.sum(cen * cen, axis=1)[:, None])
    qt = 1.0 / (1.0 + d2t + 1e-8)
    qt_ref[...] = qt / jnp.sum(qt, axis=0, keepdims=True)


_tc_c = pl.pallas_call(
    _tc_c_body,
    out_shape=[
        jax.ShapeDtypeStruct((D_E, N), jnp.float32),
        jax.ShapeDtypeStruct((N, D_IN), jnp.float32),
        jax.ShapeDtypeStruct((K, N), jnp.float32),
    ],
)


def kernel(x, edge_index, W_gat, attn_l, attn_r, b_gat, W_enc, W_dec,
           centroids):
    ei3 = edge_index.reshape(2, NBLOCKS, BL).transpose(1, 0, 2)
    feat, el, er, m16 = _tc_a(x, W_gat, attn_l.reshape(1, D_H),
                              attn_r.reshape(1, D_H))
    ee, den_parts = _sc_attn(ei3, el, er, m16)
    denom = _tc_red(den_parts)
    parts = _sc_aggr(ei3, ee, denom, feat)
    bg2 = jnp.concatenate([b_gat, b_gat]).reshape(1, 2 * D_H)
    embt, pred_gene, qt = _tc_c(parts.reshape(NC, N // 2, 2 * D_H), bg2,
                                W_enc, W_dec, centroids)
    return (embt.T, pred_gene, qt.T)
